# baseline (speedup 1.0000x reference)
import functools

import jax
import jax.numpy as jnp
from jax import lax
from jax.experimental import pallas as pl
from jax.experimental.pallas import tpu as pltpu
from jax.experimental.pallas import tpu_sc as plsc

NC = 2
NS = 16
NW = NC * NS

D = 64
CH = 128


def _sc_embed(n_total: int):
    per_w = n_total // NW
    n_chunks = per_w // CH
    assert n_chunks % 2 == 0
    mesh = plsc.VectorSubcoreMesh(core_axis_name="c", subcore_axis_name="s")

    @functools.partial(
        pl.kernel,
        out_type=jax.ShapeDtypeStruct((n_total, D), jnp.float32),
        mesh=mesh,
        compiler_params=pltpu.CompilerParams(use_tc_tiling_on_sc=False),
        scratch_types=[
            pltpu.VMEM((2, CH), jnp.int32),
            pltpu.VMEM((2, CH), jnp.int32),
            pltpu.VMEM((2, CH), jnp.int32),
            pltpu.VMEM((2, CH, D), jnp.float32),
            pltpu.SemaphoreType.DMA((2,)),
            pltpu.SemaphoreType.DMA((2,)),
            pltpu.SemaphoreType.DMA((2,)),
            pltpu.SemaphoreType.DMA((2,)),
        ],
    )
    def k(word_h, head_h, tail_h, wtab_h, htab_h, ttab_h, out_h,
          idxw, idxh, idxt, bufw, semi, semw, sema, semo):
        wid = lax.axis_index("s") * NC + lax.axis_index("c")
        w_base = wid * per_w

        def start_idx(g, b):
            base = w_base + g * CH
            pltpu.async_copy(word_h.at[pl.ds(base, CH)], idxw.at[b], semi.at[b])
            pltpu.async_copy(head_h.at[pl.ds(base, CH)], idxh.at[b], semi.at[b])
            pltpu.async_copy(tail_h.at[pl.ds(base, CH)], idxt.at[b], semi.at[b])

        def wait_idx(g, b):
            base = w_base + g * CH
            pltpu.make_async_copy(word_h.at[pl.ds(base, CH)], idxw.at[b], semi.at[b]).wait()
            pltpu.make_async_copy(head_h.at[pl.ds(base, CH)], idxh.at[b], semi.at[b]).wait()
            pltpu.make_async_copy(tail_h.at[pl.ds(base, CH)], idxt.at[b], semi.at[b]).wait()

        def start_word_gather(b):
            pltpu.async_copy(wtab_h.at[idxw.at[b]], bufw.at[b], semw.at[b])

        def wait_word_gather(b):
            pltpu.make_async_copy(wtab_h.at[idxw.at[b]], bufw.at[b], semw.at[b]).wait()

        def start_store(g, b):
            base = w_base + g * CH
            pltpu.async_copy(bufw.at[b], out_h.at[pl.ds(base, CH)], semo.at[b])

        def wait_store(g, b):
            base = w_base + g * CH
            pltpu.make_async_copy(bufw.at[b], out_h.at[pl.ds(base, CH)], semo.at[b]).wait()

        start_idx(0, 0)
        wait_idx(0, 0)
        start_word_gather(0)
        start_idx(1, 1)

        def iter_body(g, b):
            b2 = 1 - b

            @pl.when(g > 0)
            def _():
                wait_store(g - 1, b2)

            @pl.when(g < n_chunks - 1)
            def _():
                wait_idx(g + 1, b2)
                start_word_gather(b2)

            wait_word_gather(b)

            ch = pltpu.async_copy(htab_h.at[idxh.at[b]], bufw.at[b], sema.at[b], add=True)
            ch.wait()
            ct = pltpu.async_copy(ttab_h.at[idxt.at[b]], bufw.at[b], sema.at[b], add=True)

            @pl.when(g < n_chunks - 2)
            def _():
                start_idx(g + 2, b)

            ct.wait()
            start_store(g, b)

        def pair_body(g2, _):
            iter_body(g2 * 2, 0)
            iter_body(g2 * 2 + 1, 1)
            return 0

        lax.fori_loop(0, n_chunks // 2, pair_body, 0)
        wait_store(n_chunks - 1, 1)

    return k


def kernel(word, head, tail, wordEmbed, headPosEmbed, tailPosEmbed):
    b, l = word.shape
    n = b * l
    wf = word.reshape(n).astype(jnp.int32)
    hf = head.reshape(n).astype(jnp.int32)
    tf = tail.reshape(n).astype(jnp.int32)
    out = _sc_embed(n)(wf, hf, tf, wordEmbed, headPosEmbed, tailPosEmbed)
    return out.reshape(b, l, D)


# speedup vs baseline: 1.2579x; 1.2579x over previous
import functools

import jax
import jax.numpy as jnp
from jax import lax
from jax.experimental import pallas as pl
from jax.experimental.pallas import tpu as pltpu
from jax.experimental.pallas import tpu_sc as plsc

NC = 2
NS = 16
NW = NC * NS

D = 64
CH = 128


def _sc_embed(n_total: int):
    per_w = n_total // NW
    n_chunks = per_w // CH
    assert n_chunks % 2 == 0
    mesh = plsc.VectorSubcoreMesh(core_axis_name="c", subcore_axis_name="s")

    @functools.partial(
        pl.kernel,
        out_type=jax.ShapeDtypeStruct((n_total, D), jnp.float32),
        mesh=mesh,
        compiler_params=pltpu.CompilerParams(use_tc_tiling_on_sc=False),
        scratch_types=[
            pltpu.VMEM((2, CH), jnp.int32),
            pltpu.VMEM((2, CH), jnp.int32),
            pltpu.VMEM((2, CH), jnp.int32),
            pltpu.VMEM((2, CH, D), jnp.float32),
            pltpu.VMEM_SHARED((512, D), jnp.float32),
            pltpu.VMEM_SHARED((512, D), jnp.float32),
            pltpu.SemaphoreType.DMA((2,)),
            pltpu.SemaphoreType.DMA((2,)),
            pltpu.SemaphoreType.DMA((2,)),
            pltpu.SemaphoreType.DMA((2,)),
        ],
    )
    def k(word_h, head_h, tail_h, wtab_h, htab_h, ttab_h, out_h,
          idxw, idxh, idxt, bufw, htab_s, ttab_s, semi, semw, sema, semo):
        wid = lax.axis_index("s") * NC + lax.axis_index("c")
        w_base = wid * per_w

        @pl.when(lax.axis_index("s") == 0)
        def _():
            pltpu.sync_copy(htab_h, htab_s)
            pltpu.sync_copy(ttab_h, ttab_s)

        plsc.subcore_barrier()

        def start_idx(g, b):
            base = w_base + g * CH
            pltpu.async_copy(word_h.at[pl.ds(base, CH)], idxw.at[b], semi.at[b])
            pltpu.async_copy(head_h.at[pl.ds(base, CH)], idxh.at[b], semi.at[b])
            pltpu.async_copy(tail_h.at[pl.ds(base, CH)], idxt.at[b], semi.at[b])

        def wait_idx(g, b):
            base = w_base + g * CH
            pltpu.make_async_copy(word_h.at[pl.ds(base, CH)], idxw.at[b], semi.at[b]).wait()
            pltpu.make_async_copy(head_h.at[pl.ds(base, CH)], idxh.at[b], semi.at[b]).wait()
            pltpu.make_async_copy(tail_h.at[pl.ds(base, CH)], idxt.at[b], semi.at[b]).wait()

        def start_word_gather(b):
            pltpu.async_copy(wtab_h.at[idxw.at[b]], bufw.at[b], semw.at[b])

        def wait_word_gather(b):
            pltpu.make_async_copy(wtab_h.at[idxw.at[b]], bufw.at[b], semw.at[b]).wait()

        def start_store(g, b):
            base = w_base + g * CH
            pltpu.async_copy(bufw.at[b], out_h.at[pl.ds(base, CH)], semo.at[b])

        def wait_store(g, b):
            base = w_base + g * CH
            pltpu.make_async_copy(bufw.at[b], out_h.at[pl.ds(base, CH)], semo.at[b]).wait()

        start_idx(0, 0)
        wait_idx(0, 0)
        start_word_gather(0)
        start_idx(1, 1)

        def iter_body(g, b):
            b2 = 1 - b

            @pl.when(g > 0)
            def _():
                wait_store(g - 1, b2)

            @pl.when(g < n_chunks - 1)
            def _():
                wait_idx(g + 1, b2)
                start_word_gather(b2)

            wait_word_gather(b)

            ch = pltpu.async_copy(htab_s.at[idxh.at[b]], bufw.at[b], sema.at[b], add=True)
            ch.wait()
            ct = pltpu.async_copy(ttab_s.at[idxt.at[b]], bufw.at[b], sema.at[b], add=True)

            @pl.when(g < n_chunks - 2)
            def _():
                start_idx(g + 2, b)

            ct.wait()
            start_store(g, b)

        def pair_body(g2, _):
            iter_body(g2 * 2, 0)
            iter_body(g2 * 2 + 1, 1)
            return 0

        lax.fori_loop(0, n_chunks // 2, pair_body, 0)
        wait_store(n_chunks - 1, 1)

    return k


def kernel(word, head, tail, wordEmbed, headPosEmbed, tailPosEmbed):
    b, l = word.shape
    n = b * l
    wf = word.reshape(n).astype(jnp.int32)
    hf = head.reshape(n).astype(jnp.int32)
    tf = tail.reshape(n).astype(jnp.int32)
    out = _sc_embed(n)(wf, hf, tf, wordEmbed, headPosEmbed, tailPosEmbed)
    return out.reshape(b, l, D)
